# baseline (speedup 1.0000x reference)
import functools

import jax
import jax.numpy as jnp
from jax import lax
from jax.experimental import pallas as pl
from jax.experimental.pallas import tpu as pltpu
from jax.experimental.pallas import tpu_sc as plsc

NC, NS, LANES = 2, 16, 16
NW = NC * NS
N_PAD = 10240
D = 128
CHUNK = 64
NBUF = 4
STAGE_CH = 16
ROWS_PER_TILE = N_PAD // NS
ZROWS = 64
K_SPLIT0 = 320
K_SPLIT1 = 32

_SC_PARAMS = pltpu.CompilerParams(needs_layout_passes=False)


def _mesh():
    return plsc.VectorSubcoreMesh(core_axis_name="c", subcore_axis_name="s",
                                  num_cores=NC, num_subcores=NS)


def _agg_body(k0, k1, x_hbm, src_hbm, dst_hbm, part_hbm,
              src_idx, dst_idx, r0, r1, r2, r3, zbuf, acc,
              g0, g1, g2, g3):
    c = lax.axis_index("c")
    s = lax.axis_index("s")
    rows = (r0, r1, r2, r3)
    sems = (g0, g1, g2, g3)

    def stage_idx(base, h):
        pltpu.sync_copy(
            src_hbm.at[pl.ds(base + h * STAGE_CH, STAGE_CH)], src_idx)
        pltpu.sync_copy(
            dst_hbm.at[pl.ds(base + h * STAGE_CH, STAGE_CH)], dst_idx)

    def fire(b, g):
        pltpu.async_copy(x_hbm.at[src_idx.at[g]], rows[b], sems[b])

    def prefire(kc, base):
        if kc == 0:
            return
        stage_idx(base, 0)
        for b in range(NBUF):
            fire(b, b)

    @pl.when(c == 0)
    def _():
        prefire(k0, s * k0)
    @pl.when(c == 1)
    def _():
        prefire(k1, NS * k0 + s * k1)

    with jax.named_scope("agg_zero"):
        def zinit(i, _):
            for j in range(D // LANES):
                zbuf[i, pl.ds(j * LANES, LANES)] = jnp.zeros(
                    (LANES,), jnp.float32)
            return 0
        lax.fori_loop(0, ZROWS, zinit, 0)
        row0 = s * ROWS_PER_TILE
        for j in range(ROWS_PER_TILE // ZROWS):
            pltpu.sync_copy(zbuf, acc.at[pl.ds(row0 + j * ZROWS, ZROWS)])
        plsc.subcore_barrier()

    def edge_phase(kc, base):
        if kc == 0:
            return
        for h in range(kc // STAGE_CH):
            if h > 0:
                stage_idx(base, h)
                for b in range(NBUF):
                    fire(b, b)
            def ring(go, _):
                for b in range(NBUF):
                    g = go * NBUF + b
                    pltpu.make_async_copy(
                        x_hbm.at[src_idx.at[g]], rows[b], sems[b]).wait()
                    pltpu.sync_copy(rows[b], acc.at[dst_idx.at[g]], add=True)
                    @pl.when(go < STAGE_CH // NBUF - 1)
                    def _():
                        fire(b, g + NBUF)
                return 0
            lax.fori_loop(0, STAGE_CH // NBUF, ring, 0)

    with jax.named_scope("agg_edges"):
        @pl.when(c == 0)
        def _():
            edge_phase(k0, s * k0)
        @pl.when(c == 1)
        def _():
            edge_phase(k1, NS * k0 + s * k1)
        plsc.subcore_barrier()

    with jax.named_scope("agg_out"):
        for j in range(ROWS_PER_TILE // D):
            r = s * ROWS_PER_TILE + j * D
            pltpu.sync_copy(acc.at[pl.ds(r, D)], part_hbm.at[c, pl.ds(r, D)])


def _build_agg(k0, k1):
    scratch = [
        pltpu.VMEM((STAGE_CH, CHUNK), jnp.int32),
        pltpu.VMEM((STAGE_CH, CHUNK), jnp.int32),
        pltpu.VMEM((CHUNK, D), jnp.float32),
        pltpu.VMEM((CHUNK, D), jnp.float32),
        pltpu.VMEM((CHUNK, D), jnp.float32),
        pltpu.VMEM((CHUNK, D), jnp.float32),
        pltpu.VMEM((ZROWS, D), jnp.float32),
        pltpu.VMEM_SHARED((N_PAD, D), jnp.float32),
        pltpu.SemaphoreType.DMA,
        pltpu.SemaphoreType.DMA,
        pltpu.SemaphoreType.DMA,
        pltpu.SemaphoreType.DMA,
    ]
    return pl.kernel(
        functools.partial(_agg_body, k0, k1),
        out_type=(jax.ShapeDtypeStruct((NC, N_PAD, D), jnp.float32),),
        mesh=_mesh(),
        scratch_types=scratch,
        compiler_params=_SC_PARAMS,
        name="sage_agg_sc",
    )


def _cnt_body(dst_hbm, cnt_hbm, dst_idx, cnt_local):
    c = lax.axis_index("c")
    s = lax.axis_index("s")
    k_chunks = dst_hbm.shape[0] // NW
    chunk = dst_hbm.shape[1]
    wid = c * NS + s

    def cinit(i, _):
        for j in range(D // LANES):
            cnt_local[pl.ds(i * D + j * LANES, LANES)] = jnp.zeros(
                (LANES,), jnp.float32)
        return 0
    lax.fori_loop(0, N_PAD // D, cinit, 0)

    pltpu.sync_copy(dst_hbm.at[pl.ds(wid * k_chunks, k_chunks)], dst_idx)
    ones16 = jnp.ones((LANES,), jnp.float32)
    def cbody(k, _):
        for j in range(chunk // LANES):
            idxv = dst_idx[k, pl.ds(j * LANES, LANES)]
            plsc.addupdate_scatter(cnt_local, [idxv], ones16)
        return 0
    lax.fori_loop(0, k_chunks, cbody, 0)
    pltpu.sync_copy(cnt_local, cnt_hbm.at[pl.ds(wid * N_PAD, N_PAD)])


def _build_cnt(k_chunks):
    scratch = [
        pltpu.VMEM((k_chunks, CHUNK), jnp.int32),
        pltpu.VMEM((N_PAD,), jnp.float32),
    ]
    return pl.kernel(
        _cnt_body,
        out_type=(jax.ShapeDtypeStruct((NW * N_PAD,), jnp.float32),),
        mesh=_mesh(),
        scratch_types=scratch,
        compiler_params=_SC_PARAMS,
        name="sage_cnt_sc",
    )


def _tc_body(relu, p_ref, recip_ref, x_ref, wl_ref, wr_ref, b_ref, o_ref):
    mean = (p_ref[0] + p_ref[1]) * recip_ref[...]
    out = (jnp.dot(mean, wl_ref[...], preferred_element_type=jnp.float32)
           + jnp.dot(x_ref[...], wr_ref[...], preferred_element_type=jnp.float32)
           + b_ref[...])
    if relu:
        out = jnp.maximum(out, 0.0)
    o_ref[...] = out


def _tc_layer(part, recip, x, W_l, W_r, b, relu):
    BT = 1024
    return pl.pallas_call(
        functools.partial(_tc_body, relu),
        grid=(N_PAD // BT,),
        in_specs=[
            pl.BlockSpec((NC, BT, D), lambda i: (0, i, 0)),
            pl.BlockSpec((BT, 1), lambda i: (i, 0)),
            pl.BlockSpec((BT, D), lambda i: (i, 0)),
            pl.BlockSpec((D, D), lambda i: (0, 0)),
            pl.BlockSpec((D, D), lambda i: (0, 0)),
            pl.BlockSpec((1, D), lambda i: (0, 0)),
        ],
        out_specs=pl.BlockSpec((BT, D), lambda i: (i, 0)),
        out_shape=jax.ShapeDtypeStruct((N_PAD, D), jnp.float32),
        name=f"sage_dense_tc_{int(relu)}",
    )(part, recip, x, W_l, W_r, b.reshape(1, D))


def kernel(x, edge_index, W1_l, W1_r, b1, W2_l, W2_r, b2):
    n, d = x.shape
    e = edge_index.shape[1]
    src = edge_index[0].astype(jnp.int32)
    dst = edge_index[1].astype(jnp.int32)

    k_min = -(-e // (NS * CHUNK))
    k_min = -(-k_min // STAGE_CH) * STAGE_CH
    k0 = K_SPLIT0
    k1 = max(K_SPLIT1, k_min - k0)
    k_sum = k0 + k1
    k_chunks = k_sum // 2
    e_pad = NS * k_sum * CHUNK
    src_p = jnp.concatenate([src, jnp.zeros((e_pad - e,), jnp.int32)])
    dst_p = jnp.concatenate(
        [dst, jnp.full((e_pad - e,), N_PAD - 1, jnp.int32)])
    src2d = src_p.reshape(NS * k_sum, CHUNK)
    dst2d = dst_p.reshape(NS * k_sum, CHUNK)
    x_pad = jnp.pad(x, ((0, N_PAD - n), (0, 0)))

    agg = _build_agg(k0, k1)
    cntk = _build_cnt(k_chunks)

    (cnt_parts,) = cntk(dst2d)
    cnt = cnt_parts.reshape(NW, N_PAD).sum(axis=0)
    recip = (1.0 / jnp.maximum(cnt, 1.0)).reshape(N_PAD, 1)

    (part1,) = agg(x_pad, src2d, dst2d)
    h = _tc_layer(part1, recip, x_pad, W1_l, W1_r, b1, relu=True)
    (part2,) = agg(h, src2d, dst2d)
    out = _tc_layer(part2, recip, h, W2_l, W2_r, b2, relu=False)
    return out[:n, :]


# speedup vs baseline: 6.4757x; 6.4757x over previous
import functools

import jax
import jax.numpy as jnp
from jax import lax
from jax.experimental import pallas as pl
from jax.experimental.pallas import tpu as pltpu
from jax.experimental.pallas import tpu_sc as plsc

NC, NS, LANES = 2, 16, 16
NW = NC * NS
N_PAD = 10240
D = 128
CHUNK = 64
NBUF = 4
STAGE_CH = 16
ROWS_PER_TILE = N_PAD // NS
ZROWS = 64
K_SPLIT0 = 320
K_SPLIT1 = 0

_SC_PARAMS = pltpu.CompilerParams(needs_layout_passes=False)


def _mesh():
    return plsc.VectorSubcoreMesh(core_axis_name="c", subcore_axis_name="s",
                                  num_cores=NC, num_subcores=NS)


def _agg_body(k0, k1, x_hbm, src_hbm, dst_hbm, part_hbm,
              src_idx, dst_idx, r0, r1, r2, r3, zbuf, acc,
              g0, g1, g2, g3):
    c = lax.axis_index("c")
    s = lax.axis_index("s")
    rows = (r0, r1, r2, r3)
    sems = (g0, g1, g2, g3)

    def stage_idx(base, h):
        pltpu.sync_copy(
            src_hbm.at[pl.ds(base + h * STAGE_CH, STAGE_CH)], src_idx)
        pltpu.sync_copy(
            dst_hbm.at[pl.ds(base + h * STAGE_CH, STAGE_CH)], dst_idx)

    def fire(b, g):
        pltpu.async_copy(x_hbm.at[src_idx.at[g]], rows[b], sems[b])

    def prefire(kc, base):
        if kc == 0:
            return
        stage_idx(base, 0)
        for b in range(NBUF):
            fire(b, b)

    @pl.when(c == 0)
    def _():
        prefire(k0, s * k0)
    @pl.when(c == 1)
    def _():
        prefire(k1, NS * k0 + s * k1)

    with jax.named_scope("agg_zero"):
        def zinit(i, _):
            for j in range(D // LANES):
                zbuf[i, pl.ds(j * LANES, LANES)] = jnp.zeros(
                    (LANES,), jnp.float32)
            return 0
        lax.fori_loop(0, ZROWS, zinit, 0)
        row0 = s * ROWS_PER_TILE
        for j in range(ROWS_PER_TILE // ZROWS):
            pltpu.sync_copy(zbuf, acc.at[pl.ds(row0 + j * ZROWS, ZROWS)])
        plsc.subcore_barrier()

    def edge_phase(kc, base):
        if kc == 0:
            return
        for h in range(kc // STAGE_CH):
            if h > 0:
                stage_idx(base, h)
                for b in range(NBUF):
                    fire(b, b)
            def ring(go, _):
                for b in range(NBUF):
                    g = go * NBUF + b
                    pltpu.make_async_copy(
                        x_hbm.at[src_idx.at[g]], rows[b], sems[b]).wait()
                    pltpu.sync_copy(rows[b], acc.at[dst_idx.at[g]], add=True)
                    @pl.when(go < STAGE_CH // NBUF - 1)
                    def _():
                        fire(b, g + NBUF)
                return 0
            lax.fori_loop(0, STAGE_CH // NBUF, ring, 0)

    with jax.named_scope("agg_edges"):
        @pl.when(c == 0)
        def _():
            edge_phase(k0, s * k0)
        @pl.when(c == 1)
        def _():
            edge_phase(k1, NS * k0 + s * k1)
        plsc.subcore_barrier()

    with jax.named_scope("agg_out"):
        for j in range(ROWS_PER_TILE // D):
            r = s * ROWS_PER_TILE + j * D
            pltpu.sync_copy(acc.at[pl.ds(r, D)], part_hbm.at[c, pl.ds(r, D)])


def _build_agg(k0, k1):
    scratch = [
        pltpu.VMEM((STAGE_CH, CHUNK), jnp.int32),
        pltpu.VMEM((STAGE_CH, CHUNK), jnp.int32),
        pltpu.VMEM((CHUNK, D), jnp.float32),
        pltpu.VMEM((CHUNK, D), jnp.float32),
        pltpu.VMEM((CHUNK, D), jnp.float32),
        pltpu.VMEM((CHUNK, D), jnp.float32),
        pltpu.VMEM((ZROWS, D), jnp.float32),
        pltpu.VMEM_SHARED((N_PAD, D), jnp.float32),
        pltpu.SemaphoreType.DMA,
        pltpu.SemaphoreType.DMA,
        pltpu.SemaphoreType.DMA,
        pltpu.SemaphoreType.DMA,
    ]
    return pl.kernel(
        functools.partial(_agg_body, k0, k1),
        out_type=(jax.ShapeDtypeStruct((NC, N_PAD, D), jnp.float32),),
        mesh=_mesh(),
        scratch_types=scratch,
        compiler_params=_SC_PARAMS,
        name="sage_agg_sc",
    )


def _cnt_body(dst_hbm, cnt_hbm, dst_idx, cnt_local):
    c = lax.axis_index("c")
    s = lax.axis_index("s")
    k_chunks = dst_hbm.shape[0] // NW
    chunk = dst_hbm.shape[1]
    wid = c * NS + s

    def cinit(i, _):
        for j in range(D // LANES):
            cnt_local[pl.ds(i * D + j * LANES, LANES)] = jnp.zeros(
                (LANES,), jnp.float32)
        return 0
    lax.fori_loop(0, N_PAD // D, cinit, 0)

    pltpu.sync_copy(dst_hbm.at[pl.ds(wid * k_chunks, k_chunks)], dst_idx)
    ones16 = jnp.ones((LANES,), jnp.float32)
    def cbody(k, _):
        for j in range(chunk // LANES):
            idxv = dst_idx[k, pl.ds(j * LANES, LANES)]
            plsc.addupdate_scatter(cnt_local, [idxv], ones16)
        return 0
    lax.fori_loop(0, k_chunks, cbody, 0)
    pltpu.sync_copy(cnt_local, cnt_hbm.at[pl.ds(wid * N_PAD, N_PAD)])


def _build_cnt(k_chunks):
    scratch = [
        pltpu.VMEM((k_chunks, CHUNK), jnp.int32),
        pltpu.VMEM((N_PAD,), jnp.float32),
    ]
    return pl.kernel(
        _cnt_body,
        out_type=(jax.ShapeDtypeStruct((NW * N_PAD,), jnp.float32),),
        mesh=_mesh(),
        scratch_types=scratch,
        compiler_params=_SC_PARAMS,
        name="sage_cnt_sc",
    )


def _tc_body(relu, p_ref, recip_ref, x_ref, wl_ref, wr_ref, b_ref, o_ref):
    mean = (p_ref[0] + p_ref[1]) * recip_ref[...]
    out = (jnp.dot(mean, wl_ref[...], preferred_element_type=jnp.float32)
           + jnp.dot(x_ref[...], wr_ref[...], preferred_element_type=jnp.float32)
           + b_ref[...])
    if relu:
        out = jnp.maximum(out, 0.0)
    o_ref[...] = out


def _tc_layer(part, recip, x, W_l, W_r, b, relu):
    BT = 1024
    return pl.pallas_call(
        functools.partial(_tc_body, relu),
        grid=(N_PAD // BT,),
        in_specs=[
            pl.BlockSpec((NC, BT, D), lambda i: (0, i, 0)),
            pl.BlockSpec((BT, 1), lambda i: (i, 0)),
            pl.BlockSpec((BT, D), lambda i: (i, 0)),
            pl.BlockSpec((D, D), lambda i: (0, 0)),
            pl.BlockSpec((D, D), lambda i: (0, 0)),
            pl.BlockSpec((1, D), lambda i: (0, 0)),
        ],
        out_specs=pl.BlockSpec((BT, D), lambda i: (i, 0)),
        out_shape=jax.ShapeDtypeStruct((N_PAD, D), jnp.float32),
        name=f"sage_dense_tc_{int(relu)}",
    )(part, recip, x, W_l, W_r, b.reshape(1, D))


def kernel(x, edge_index, W1_l, W1_r, b1, W2_l, W2_r, b2):
    n, d = x.shape
    e = edge_index.shape[1]
    src = edge_index[0].astype(jnp.int32)
    dst = edge_index[1].astype(jnp.int32)

    k_min = -(-e // (NS * CHUNK))
    k_min = -(-k_min // STAGE_CH) * STAGE_CH
    k0 = K_SPLIT0
    k1 = max(K_SPLIT1, k_min - k0)
    k_sum = k0 + k1
    k_chunks = k_sum // 2
    e_pad = NS * k_sum * CHUNK
    pad_src = jnp.arange(e_pad - e, dtype=jnp.int32) % jnp.int32(n)
    src_p = jnp.concatenate([src, pad_src])
    dst_p = jnp.concatenate(
        [dst, jnp.full((e_pad - e,), N_PAD - 1, jnp.int32)])
    src2d = src_p.reshape(NS * k_sum, CHUNK)
    dst2d = dst_p.reshape(NS * k_sum, CHUNK)
    x_pad = jnp.pad(x, ((0, N_PAD - n), (0, 0)))

    agg = _build_agg(k0, k1)
    cntk = _build_cnt(k_chunks)

    (cnt_parts,) = cntk(dst2d)
    cnt = cnt_parts.reshape(NW, N_PAD).sum(axis=0)
    recip = (1.0 / jnp.maximum(cnt, 1.0)).reshape(N_PAD, 1)

    (part1,) = agg(x_pad, src2d, dst2d)
    h = _tc_layer(part1, recip, x_pad, W1_l, W1_r, b1, relu=True)
    (part2,) = agg(h, src2d, dst2d)
    out = _tc_layer(part2, recip, h, W2_l, W2_r, b2, relu=False)
    return out[:n, :]


# speedup vs baseline: 7.0514x; 1.0889x over previous
import functools

import jax
import jax.numpy as jnp
from jax import lax
from jax.experimental import pallas as pl
from jax.experimental.pallas import tpu as pltpu
from jax.experimental.pallas import tpu_sc as plsc

NC, NS, LANES = 2, 16, 16
NW = NC * NS
N_PAD = 10240
D = 128
CHUNK = 64
NBUF = 4
STAGE_CH = 16
ROWS_PER_TILE = N_PAD // NS
ZROWS = 64
K_SPLIT0 = 288
K_SPLIT1 = 0

_SC_PARAMS = pltpu.CompilerParams(needs_layout_passes=False)


def _mesh():
    return plsc.VectorSubcoreMesh(core_axis_name="c", subcore_axis_name="s",
                                  num_cores=NC, num_subcores=NS)


def _agg_body(k0, k1, x_hbm, src_hbm, dst_hbm, part_hbm,
              src_idx, dst_idx, r0, r1, r2, r3, zbuf, acc,
              g0, g1, g2, g3):
    c = lax.axis_index("c")
    s = lax.axis_index("s")
    rows = (r0, r1, r2, r3)
    sems = (g0, g1, g2, g3)

    def stage_idx(base, h):
        pltpu.sync_copy(
            src_hbm.at[pl.ds(base + h * STAGE_CH, STAGE_CH)], src_idx)
        pltpu.sync_copy(
            dst_hbm.at[pl.ds(base + h * STAGE_CH, STAGE_CH)], dst_idx)

    def fire(b, g):
        pltpu.async_copy(x_hbm.at[src_idx.at[g]], rows[b], sems[b])

    def prefire(kc, base):
        if kc == 0:
            return
        stage_idx(base, 0)
        for b in range(NBUF):
            fire(b, b)

    @pl.when(c == 0)
    def _():
        prefire(k0, s * k0)
    @pl.when(c == 1)
    def _():
        prefire(k1, NS * k0 + s * k1)

    with jax.named_scope("agg_zero"):
        def zinit(i, _):
            for j in range(D // LANES):
                zbuf[i, pl.ds(j * LANES, LANES)] = jnp.zeros(
                    (LANES,), jnp.float32)
            return 0
        lax.fori_loop(0, ZROWS, zinit, 0)
        row0 = s * ROWS_PER_TILE
        for j in range(ROWS_PER_TILE // ZROWS):
            pltpu.sync_copy(zbuf, acc.at[pl.ds(row0 + j * ZROWS, ZROWS)])
        plsc.subcore_barrier()

    def edge_phase(kc, base):
        if kc == 0:
            return
        for h in range(kc // STAGE_CH):
            if h > 0:
                stage_idx(base, h)
                for b in range(NBUF):
                    fire(b, b)
            def ring(go, _):
                for b in range(NBUF):
                    g = go * NBUF + b
                    pltpu.make_async_copy(
                        x_hbm.at[src_idx.at[g]], rows[b], sems[b]).wait()
                    pltpu.sync_copy(rows[b], acc.at[dst_idx.at[g]], add=True)
                    @pl.when(go < STAGE_CH // NBUF - 1)
                    def _():
                        fire(b, g + NBUF)
                return 0
            lax.fori_loop(0, STAGE_CH // NBUF, ring, 0)

    with jax.named_scope("agg_edges"):
        @pl.when(c == 0)
        def _():
            edge_phase(k0, s * k0)
        @pl.when(c == 1)
        def _():
            edge_phase(k1, NS * k0 + s * k1)
        plsc.subcore_barrier()

    with jax.named_scope("agg_out"):
        for j in range(ROWS_PER_TILE // D):
            r = s * ROWS_PER_TILE + j * D
            pltpu.sync_copy(acc.at[pl.ds(r, D)], part_hbm.at[c, pl.ds(r, D)])


def _build_agg(k0, k1):
    scratch = [
        pltpu.VMEM((STAGE_CH, CHUNK), jnp.int32),
        pltpu.VMEM((STAGE_CH, CHUNK), jnp.int32),
        pltpu.VMEM((CHUNK, D), jnp.float32),
        pltpu.VMEM((CHUNK, D), jnp.float32),
        pltpu.VMEM((CHUNK, D), jnp.float32),
        pltpu.VMEM((CHUNK, D), jnp.float32),
        pltpu.VMEM((ZROWS, D), jnp.float32),
        pltpu.VMEM_SHARED((N_PAD, D), jnp.float32),
        pltpu.SemaphoreType.DMA,
        pltpu.SemaphoreType.DMA,
        pltpu.SemaphoreType.DMA,
        pltpu.SemaphoreType.DMA,
    ]
    return pl.kernel(
        functools.partial(_agg_body, k0, k1),
        out_type=(jax.ShapeDtypeStruct((NC, N_PAD, D), jnp.float32),),
        mesh=_mesh(),
        scratch_types=scratch,
        compiler_params=_SC_PARAMS,
        name="sage_agg_sc",
    )


def _cnt_body(dst_hbm, cnt_hbm, dst_idx, cnt_local):
    c = lax.axis_index("c")
    s = lax.axis_index("s")
    k_chunks = dst_hbm.shape[0] // NW
    chunk = dst_hbm.shape[1]
    wid = c * NS + s

    def cinit(i, _):
        for j in range(D // LANES):
            cnt_local[pl.ds(i * D + j * LANES, LANES)] = jnp.zeros(
                (LANES,), jnp.float32)
        return 0
    lax.fori_loop(0, N_PAD // D, cinit, 0)

    pltpu.sync_copy(dst_hbm.at[pl.ds(wid * k_chunks, k_chunks)], dst_idx)
    ones16 = jnp.ones((LANES,), jnp.float32)
    def cbody(k, _):
        for j in range(chunk // LANES):
            idxv = dst_idx[k, pl.ds(j * LANES, LANES)]
            plsc.addupdate_scatter(cnt_local, [idxv], ones16)
        return 0
    lax.fori_loop(0, k_chunks, cbody, 0)
    pltpu.sync_copy(cnt_local, cnt_hbm.at[pl.ds(wid * N_PAD, N_PAD)])


def _build_cnt(k_chunks):
    scratch = [
        pltpu.VMEM((k_chunks, CHUNK), jnp.int32),
        pltpu.VMEM((N_PAD,), jnp.float32),
    ]
    return pl.kernel(
        _cnt_body,
        out_type=(jax.ShapeDtypeStruct((NW * N_PAD,), jnp.float32),),
        mesh=_mesh(),
        scratch_types=scratch,
        compiler_params=_SC_PARAMS,
        name="sage_cnt_sc",
    )


def _tc_body(relu, p_ref, recip_ref, x_ref, wl_ref, wr_ref, b_ref, o_ref):
    mean = (p_ref[0] + p_ref[1]) * recip_ref[...]
    out = (jnp.dot(mean, wl_ref[...], preferred_element_type=jnp.float32)
           + jnp.dot(x_ref[...], wr_ref[...], preferred_element_type=jnp.float32)
           + b_ref[...])
    if relu:
        out = jnp.maximum(out, 0.0)
    o_ref[...] = out


def _tc_layer(part, recip, x, W_l, W_r, b, relu):
    BT = 1024
    return pl.pallas_call(
        functools.partial(_tc_body, relu),
        grid=(N_PAD // BT,),
        in_specs=[
            pl.BlockSpec((NC, BT, D), lambda i: (0, i, 0)),
            pl.BlockSpec((BT, 1), lambda i: (i, 0)),
            pl.BlockSpec((BT, D), lambda i: (i, 0)),
            pl.BlockSpec((D, D), lambda i: (0, 0)),
            pl.BlockSpec((D, D), lambda i: (0, 0)),
            pl.BlockSpec((1, D), lambda i: (0, 0)),
        ],
        out_specs=pl.BlockSpec((BT, D), lambda i: (i, 0)),
        out_shape=jax.ShapeDtypeStruct((N_PAD, D), jnp.float32),
        name=f"sage_dense_tc_{int(relu)}",
    )(part, recip, x, W_l, W_r, b.reshape(1, D))


def kernel(x, edge_index, W1_l, W1_r, b1, W2_l, W2_r, b2):
    n, d = x.shape
    e = edge_index.shape[1]
    src = edge_index[0].astype(jnp.int32)
    dst = edge_index[1].astype(jnp.int32)

    k_min = -(-e // (NS * CHUNK))
    k_min = -(-k_min // STAGE_CH) * STAGE_CH
    k0 = K_SPLIT0
    k1 = max(K_SPLIT1, k_min - k0)
    k_sum = k0 + k1
    k_chunks = k_sum // 2
    e_pad = NS * k_sum * CHUNK
    pad_src = jnp.arange(e_pad - e, dtype=jnp.int32) % jnp.int32(n)
    src_p = jnp.concatenate([src, pad_src])
    dst_p = jnp.concatenate(
        [dst, jnp.full((e_pad - e,), N_PAD - 1, jnp.int32)])
    src2d = src_p.reshape(NS * k_sum, CHUNK)
    dst2d = dst_p.reshape(NS * k_sum, CHUNK)
    x_pad = jnp.pad(x, ((0, N_PAD - n), (0, 0)))

    agg = _build_agg(k0, k1)
    cntk = _build_cnt(k_chunks)

    (cnt_parts,) = cntk(dst2d)
    cnt = cnt_parts.reshape(NW, N_PAD).sum(axis=0)
    recip = (1.0 / jnp.maximum(cnt, 1.0)).reshape(N_PAD, 1)

    (part1,) = agg(x_pad, src2d, dst2d)
    h = _tc_layer(part1, recip, x_pad, W1_l, W1_r, b1, relu=True)
    (part2,) = agg(h, src2d, dst2d)
    out = _tc_layer(part2, recip, h, W2_l, W2_r, b2, relu=False)
    return out[:n, :]


# speedup vs baseline: 7.7108x; 1.0935x over previous
import functools

import jax
import jax.numpy as jnp
from jax import lax
from jax.experimental import pallas as pl
from jax.experimental.pallas import tpu as pltpu
from jax.experimental.pallas import tpu_sc as plsc

NC, NS, LANES = 2, 16, 16
NW = NC * NS
N_PAD = 10240
D = 128
CHUNK = 64
NBUF = 4
STAGE_CH = 16
ROWS_PER_TILE = N_PAD // NS
ZROWS = 64
K_SPLIT0 = 256
K_SPLIT1 = 0

_SC_PARAMS = pltpu.CompilerParams(needs_layout_passes=False)


def _mesh():
    return plsc.VectorSubcoreMesh(core_axis_name="c", subcore_axis_name="s",
                                  num_cores=NC, num_subcores=NS)


def _agg_body(k0, k1, x_hbm, src_hbm, dst_hbm, part_hbm,
              src_idx, dst_idx, r0, r1, r2, r3, zbuf, acc,
              g0, g1, g2, g3):
    c = lax.axis_index("c")
    s = lax.axis_index("s")
    rows = (r0, r1, r2, r3)
    sems = (g0, g1, g2, g3)

    def stage_idx(base, h):
        pltpu.sync_copy(
            src_hbm.at[pl.ds(base + h * STAGE_CH, STAGE_CH)], src_idx)
        pltpu.sync_copy(
            dst_hbm.at[pl.ds(base + h * STAGE_CH, STAGE_CH)], dst_idx)

    def fire(b, g):
        pltpu.async_copy(x_hbm.at[src_idx.at[g]], rows[b], sems[b])

    def prefire(kc, base):
        if kc == 0:
            return
        stage_idx(base, 0)
        for b in range(NBUF):
            fire(b, b)

    @pl.when(c == 0)
    def _():
        prefire(k0, s * k0)
    @pl.when(c == 1)
    def _():
        prefire(k1, NS * k0 + s * k1)

    with jax.named_scope("agg_zero"):
        def zinit(i, _):
            for j in range(D // LANES):
                zbuf[i, pl.ds(j * LANES, LANES)] = jnp.zeros(
                    (LANES,), jnp.float32)
            return 0
        lax.fori_loop(0, ZROWS, zinit, 0)
        row0 = s * ROWS_PER_TILE
        for j in range(ROWS_PER_TILE // ZROWS):
            pltpu.sync_copy(zbuf, acc.at[pl.ds(row0 + j * ZROWS, ZROWS)])
        plsc.subcore_barrier()

    def edge_phase(kc, base):
        if kc == 0:
            return
        for h in range(kc // STAGE_CH):
            if h > 0:
                stage_idx(base, h)
                for b in range(NBUF):
                    fire(b, b)
            def ring(go, _):
                for b in range(NBUF):
                    g = go * NBUF + b
                    pltpu.make_async_copy(
                        x_hbm.at[src_idx.at[g]], rows[b], sems[b]).wait()
                    pltpu.sync_copy(rows[b], acc.at[dst_idx.at[g]], add=True)
                    @pl.when(go < STAGE_CH // NBUF - 1)
                    def _():
                        fire(b, g + NBUF)
                return 0
            lax.fori_loop(0, STAGE_CH // NBUF, ring, 0)

    with jax.named_scope("agg_edges"):
        @pl.when(c == 0)
        def _():
            edge_phase(k0, s * k0)
        @pl.when(c == 1)
        def _():
            edge_phase(k1, NS * k0 + s * k1)
        plsc.subcore_barrier()

    with jax.named_scope("agg_out"):
        for j in range(ROWS_PER_TILE // D):
            r = s * ROWS_PER_TILE + j * D
            pltpu.sync_copy(acc.at[pl.ds(r, D)], part_hbm.at[c, pl.ds(r, D)])


def _build_agg(k0, k1):
    scratch = [
        pltpu.VMEM((STAGE_CH, CHUNK), jnp.int32),
        pltpu.VMEM((STAGE_CH, CHUNK), jnp.int32),
        pltpu.VMEM((CHUNK, D), jnp.float32),
        pltpu.VMEM((CHUNK, D), jnp.float32),
        pltpu.VMEM((CHUNK, D), jnp.float32),
        pltpu.VMEM((CHUNK, D), jnp.float32),
        pltpu.VMEM((ZROWS, D), jnp.float32),
        pltpu.VMEM_SHARED((N_PAD, D), jnp.float32),
        pltpu.SemaphoreType.DMA,
        pltpu.SemaphoreType.DMA,
        pltpu.SemaphoreType.DMA,
        pltpu.SemaphoreType.DMA,
    ]
    return pl.kernel(
        functools.partial(_agg_body, k0, k1),
        out_type=(jax.ShapeDtypeStruct((NC, N_PAD, D), jnp.float32),),
        mesh=_mesh(),
        scratch_types=scratch,
        compiler_params=_SC_PARAMS,
        name="sage_agg_sc",
    )


def _cnt_body(dst_hbm, cnt_hbm, dst_idx, cnt_local):
    c = lax.axis_index("c")
    s = lax.axis_index("s")
    k_chunks = dst_hbm.shape[0] // NW
    chunk = dst_hbm.shape[1]
    wid = c * NS + s

    def cinit(i, _):
        for j in range(D // LANES):
            cnt_local[pl.ds(i * D + j * LANES, LANES)] = jnp.zeros(
                (LANES,), jnp.float32)
        return 0
    lax.fori_loop(0, N_PAD // D, cinit, 0)

    pltpu.sync_copy(dst_hbm.at[pl.ds(wid * k_chunks, k_chunks)], dst_idx)
    ones16 = jnp.ones((LANES,), jnp.float32)
    def cbody(k, _):
        for j in range(chunk // LANES):
            idxv = dst_idx[k, pl.ds(j * LANES, LANES)]
            plsc.addupdate_scatter(cnt_local, [idxv], ones16)
        return 0
    lax.fori_loop(0, k_chunks, cbody, 0)
    pltpu.sync_copy(cnt_local, cnt_hbm.at[pl.ds(wid * N_PAD, N_PAD)])


def _build_cnt(k_chunks):
    scratch = [
        pltpu.VMEM((k_chunks, CHUNK), jnp.int32),
        pltpu.VMEM((N_PAD,), jnp.float32),
    ]
    return pl.kernel(
        _cnt_body,
        out_type=(jax.ShapeDtypeStruct((NW * N_PAD,), jnp.float32),),
        mesh=_mesh(),
        scratch_types=scratch,
        compiler_params=_SC_PARAMS,
        name="sage_cnt_sc",
    )


def _tc_body(relu, p_ref, recip_ref, x_ref, wl_ref, wr_ref, b_ref, o_ref):
    mean = (p_ref[0] + p_ref[1]) * recip_ref[...]
    out = (jnp.dot(mean, wl_ref[...], preferred_element_type=jnp.float32)
           + jnp.dot(x_ref[...], wr_ref[...], preferred_element_type=jnp.float32)
           + b_ref[...])
    if relu:
        out = jnp.maximum(out, 0.0)
    o_ref[...] = out


def _tc_layer(part, recip, x, W_l, W_r, b, relu):
    BT = 1024
    return pl.pallas_call(
        functools.partial(_tc_body, relu),
        grid=(N_PAD // BT,),
        in_specs=[
            pl.BlockSpec((NC, BT, D), lambda i: (0, i, 0)),
            pl.BlockSpec((BT, 1), lambda i: (i, 0)),
            pl.BlockSpec((BT, D), lambda i: (i, 0)),
            pl.BlockSpec((D, D), lambda i: (0, 0)),
            pl.BlockSpec((D, D), lambda i: (0, 0)),
            pl.BlockSpec((1, D), lambda i: (0, 0)),
        ],
        out_specs=pl.BlockSpec((BT, D), lambda i: (i, 0)),
        out_shape=jax.ShapeDtypeStruct((N_PAD, D), jnp.float32),
        name=f"sage_dense_tc_{int(relu)}",
    )(part, recip, x, W_l, W_r, b.reshape(1, D))


def kernel(x, edge_index, W1_l, W1_r, b1, W2_l, W2_r, b2):
    n, d = x.shape
    e = edge_index.shape[1]
    src = edge_index[0].astype(jnp.int32)
    dst = edge_index[1].astype(jnp.int32)

    k_min = -(-e // (NS * CHUNK))
    k_min = -(-k_min // STAGE_CH) * STAGE_CH
    k0 = K_SPLIT0
    k1 = max(K_SPLIT1, k_min - k0)
    k_sum = k0 + k1
    k_chunks = k_sum // 2
    e_pad = NS * k_sum * CHUNK
    pad_src = jnp.arange(e_pad - e, dtype=jnp.int32) % jnp.int32(n)
    src_p = jnp.concatenate([src, pad_src])
    dst_p = jnp.concatenate(
        [dst, jnp.full((e_pad - e,), N_PAD - 1, jnp.int32)])
    src2d = src_p.reshape(NS * k_sum, CHUNK)
    dst2d = dst_p.reshape(NS * k_sum, CHUNK)
    x_pad = jnp.pad(x, ((0, N_PAD - n), (0, 0)))

    agg = _build_agg(k0, k1)
    cntk = _build_cnt(k_chunks)

    (cnt_parts,) = cntk(dst2d)
    cnt = cnt_parts.reshape(NW, N_PAD).sum(axis=0)
    recip = (1.0 / jnp.maximum(cnt, 1.0)).reshape(N_PAD, 1)

    (part1,) = agg(x_pad, src2d, dst2d)
    h = _tc_layer(part1, recip, x_pad, W1_l, W1_r, b1, relu=True)
    (part2,) = agg(h, src2d, dst2d)
    out = _tc_layer(part2, recip, h, W2_l, W2_r, b2, relu=False)
    return out[:n, :]


# speedup vs baseline: 8.5326x; 1.1066x over previous
import functools

import jax
import jax.numpy as jnp
from jax import lax
from jax.experimental import pallas as pl
from jax.experimental.pallas import tpu as pltpu
from jax.experimental.pallas import tpu_sc as plsc

NC, NS, LANES = 2, 16, 16
NW = NC * NS
N_PAD = 10240
D = 128
CHUNK = 64
NBUF = 4
STAGE_CH = 16
ROWS_PER_TILE = N_PAD // NS
ZROWS = 64
K_SPLIT0 = 224
K_SPLIT1 = 0

_SC_PARAMS = pltpu.CompilerParams(needs_layout_passes=False)


def _mesh():
    return plsc.VectorSubcoreMesh(core_axis_name="c", subcore_axis_name="s",
                                  num_cores=NC, num_subcores=NS)


def _agg_body(k0, k1, x_hbm, src_hbm, dst_hbm, part_hbm,
              src_idx, dst_idx, r0, r1, r2, r3, zbuf, acc,
              g0, g1, g2, g3):
    c = lax.axis_index("c")
    s = lax.axis_index("s")
    rows = (r0, r1, r2, r3)
    sems = (g0, g1, g2, g3)

    def stage_idx(base, h):
        pltpu.sync_copy(
            src_hbm.at[pl.ds(base + h * STAGE_CH, STAGE_CH)], src_idx)
        pltpu.sync_copy(
            dst_hbm.at[pl.ds(base + h * STAGE_CH, STAGE_CH)], dst_idx)

    def fire(b, g):
        pltpu.async_copy(x_hbm.at[src_idx.at[g]], rows[b], sems[b])

    def prefire(kc, base):
        if kc == 0:
            return
        stage_idx(base, 0)
        for b in range(NBUF):
            fire(b, b)

    @pl.when(c == 0)
    def _():
        prefire(k0, s * k0)
    @pl.when(c == 1)
    def _():
        prefire(k1, NS * k0 + s * k1)

    with jax.named_scope("agg_zero"):
        def zinit(i, _):
            for j in range(D // LANES):
                zbuf[i, pl.ds(j * LANES, LANES)] = jnp.zeros(
                    (LANES,), jnp.float32)
            return 0
        lax.fori_loop(0, ZROWS, zinit, 0)
        row0 = s * ROWS_PER_TILE
        for j in range(ROWS_PER_TILE // ZROWS):
            pltpu.sync_copy(zbuf, acc.at[pl.ds(row0 + j * ZROWS, ZROWS)])
        plsc.subcore_barrier()

    def edge_phase(kc, base):
        if kc == 0:
            return
        for h in range(kc // STAGE_CH):
            if h > 0:
                stage_idx(base, h)
                for b in range(NBUF):
                    fire(b, b)
            def ring(go, _):
                for b in range(NBUF):
                    g = go * NBUF + b
                    pltpu.make_async_copy(
                        x_hbm.at[src_idx.at[g]], rows[b], sems[b]).wait()
                    pltpu.sync_copy(rows[b], acc.at[dst_idx.at[g]], add=True)
                    @pl.when(go < STAGE_CH // NBUF - 1)
                    def _():
                        fire(b, g + NBUF)
                return 0
            lax.fori_loop(0, STAGE_CH // NBUF, ring, 0)

    with jax.named_scope("agg_edges"):
        @pl.when(c == 0)
        def _():
            edge_phase(k0, s * k0)
        @pl.when(c == 1)
        def _():
            edge_phase(k1, NS * k0 + s * k1)
        plsc.subcore_barrier()

    with jax.named_scope("agg_out"):
        for j in range(ROWS_PER_TILE // D):
            r = s * ROWS_PER_TILE + j * D
            pltpu.sync_copy(acc.at[pl.ds(r, D)], part_hbm.at[c, pl.ds(r, D)])


def _build_agg(k0, k1):
    scratch = [
        pltpu.VMEM((STAGE_CH, CHUNK), jnp.int32),
        pltpu.VMEM((STAGE_CH, CHUNK), jnp.int32),
        pltpu.VMEM((CHUNK, D), jnp.float32),
        pltpu.VMEM((CHUNK, D), jnp.float32),
        pltpu.VMEM((CHUNK, D), jnp.float32),
        pltpu.VMEM((CHUNK, D), jnp.float32),
        pltpu.VMEM((ZROWS, D), jnp.float32),
        pltpu.VMEM_SHARED((N_PAD, D), jnp.float32),
        pltpu.SemaphoreType.DMA,
        pltpu.SemaphoreType.DMA,
        pltpu.SemaphoreType.DMA,
        pltpu.SemaphoreType.DMA,
    ]
    return pl.kernel(
        functools.partial(_agg_body, k0, k1),
        out_type=(jax.ShapeDtypeStruct((NC, N_PAD, D), jnp.float32),),
        mesh=_mesh(),
        scratch_types=scratch,
        compiler_params=_SC_PARAMS,
        name="sage_agg_sc",
    )


def _cnt_body(dst_hbm, cnt_hbm, dst_idx, cnt_local):
    c = lax.axis_index("c")
    s = lax.axis_index("s")
    k_chunks = dst_hbm.shape[0] // NW
    chunk = dst_hbm.shape[1]
    wid = c * NS + s

    def cinit(i, _):
        for j in range(D // LANES):
            cnt_local[pl.ds(i * D + j * LANES, LANES)] = jnp.zeros(
                (LANES,), jnp.float32)
        return 0
    lax.fori_loop(0, N_PAD // D, cinit, 0)

    pltpu.sync_copy(dst_hbm.at[pl.ds(wid * k_chunks, k_chunks)], dst_idx)
    ones16 = jnp.ones((LANES,), jnp.float32)
    def cbody(k, _):
        for j in range(chunk // LANES):
            idxv = dst_idx[k, pl.ds(j * LANES, LANES)]
            plsc.addupdate_scatter(cnt_local, [idxv], ones16)
        return 0
    lax.fori_loop(0, k_chunks, cbody, 0)
    pltpu.sync_copy(cnt_local, cnt_hbm.at[pl.ds(wid * N_PAD, N_PAD)])


def _build_cnt(k_chunks):
    scratch = [
        pltpu.VMEM((k_chunks, CHUNK), jnp.int32),
        pltpu.VMEM((N_PAD,), jnp.float32),
    ]
    return pl.kernel(
        _cnt_body,
        out_type=(jax.ShapeDtypeStruct((NW * N_PAD,), jnp.float32),),
        mesh=_mesh(),
        scratch_types=scratch,
        compiler_params=_SC_PARAMS,
        name="sage_cnt_sc",
    )


def _tc_body(relu, p_ref, recip_ref, x_ref, wl_ref, wr_ref, b_ref, o_ref):
    mean = (p_ref[0] + p_ref[1]) * recip_ref[...]
    out = (jnp.dot(mean, wl_ref[...], preferred_element_type=jnp.float32)
           + jnp.dot(x_ref[...], wr_ref[...], preferred_element_type=jnp.float32)
           + b_ref[...])
    if relu:
        out = jnp.maximum(out, 0.0)
    o_ref[...] = out


def _tc_layer(part, recip, x, W_l, W_r, b, relu):
    BT = 1024
    return pl.pallas_call(
        functools.partial(_tc_body, relu),
        grid=(N_PAD // BT,),
        in_specs=[
            pl.BlockSpec((NC, BT, D), lambda i: (0, i, 0)),
            pl.BlockSpec((BT, 1), lambda i: (i, 0)),
            pl.BlockSpec((BT, D), lambda i: (i, 0)),
            pl.BlockSpec((D, D), lambda i: (0, 0)),
            pl.BlockSpec((D, D), lambda i: (0, 0)),
            pl.BlockSpec((1, D), lambda i: (0, 0)),
        ],
        out_specs=pl.BlockSpec((BT, D), lambda i: (i, 0)),
        out_shape=jax.ShapeDtypeStruct((N_PAD, D), jnp.float32),
        name=f"sage_dense_tc_{int(relu)}",
    )(part, recip, x, W_l, W_r, b.reshape(1, D))


def kernel(x, edge_index, W1_l, W1_r, b1, W2_l, W2_r, b2):
    n, d = x.shape
    e = edge_index.shape[1]
    src = edge_index[0].astype(jnp.int32)
    dst = edge_index[1].astype(jnp.int32)

    k_min = -(-e // (NS * CHUNK))
    k_min = -(-k_min // STAGE_CH) * STAGE_CH
    k0 = K_SPLIT0
    k1 = max(K_SPLIT1, k_min - k0)
    k_sum = k0 + k1
    k_chunks = k_sum // 2
    e_pad = NS * k_sum * CHUNK
    pad_src = jnp.arange(e_pad - e, dtype=jnp.int32) % jnp.int32(n)
    src_p = jnp.concatenate([src, pad_src])
    dst_p = jnp.concatenate(
        [dst, jnp.full((e_pad - e,), N_PAD - 1, jnp.int32)])
    src2d = src_p.reshape(NS * k_sum, CHUNK)
    dst2d = dst_p.reshape(NS * k_sum, CHUNK)
    x_pad = jnp.pad(x, ((0, N_PAD - n), (0, 0)))

    agg = _build_agg(k0, k1)
    cntk = _build_cnt(k_chunks)

    (cnt_parts,) = cntk(dst2d)
    cnt = cnt_parts.reshape(NW, N_PAD).sum(axis=0)
    recip = (1.0 / jnp.maximum(cnt, 1.0)).reshape(N_PAD, 1)

    (part1,) = agg(x_pad, src2d, dst2d)
    h = _tc_layer(part1, recip, x_pad, W1_l, W1_r, b1, relu=True)
    (part2,) = agg(h, src2d, dst2d)
    out = _tc_layer(part2, recip, h, W2_l, W2_r, b2, relu=False)
    return out[:n, :]


# speedup vs baseline: 9.5279x; 1.1166x over previous
import functools

import jax
import jax.numpy as jnp
from jax import lax
from jax.experimental import pallas as pl
from jax.experimental.pallas import tpu as pltpu
from jax.experimental.pallas import tpu_sc as plsc

NC, NS, LANES = 2, 16, 16
NW = NC * NS
N_PAD = 10240
D = 128
CHUNK = 64
NBUF = 4
STAGE_CH = 16
ROWS_PER_TILE = N_PAD // NS
ZROWS = 64
K_SPLIT0 = 192
K_SPLIT1 = 0

_SC_PARAMS = pltpu.CompilerParams(needs_layout_passes=False)


def _mesh():
    return plsc.VectorSubcoreMesh(core_axis_name="c", subcore_axis_name="s",
                                  num_cores=NC, num_subcores=NS)


def _agg_body(k0, k1, x_hbm, src_hbm, dst_hbm, part_hbm,
              src_idx, dst_idx, r0, r1, r2, r3, zbuf, acc,
              g0, g1, g2, g3):
    c = lax.axis_index("c")
    s = lax.axis_index("s")
    rows = (r0, r1, r2, r3)
    sems = (g0, g1, g2, g3)

    def stage_idx(base, h):
        pltpu.sync_copy(
            src_hbm.at[pl.ds(base + h * STAGE_CH, STAGE_CH)], src_idx)
        pltpu.sync_copy(
            dst_hbm.at[pl.ds(base + h * STAGE_CH, STAGE_CH)], dst_idx)

    def fire(b, g):
        pltpu.async_copy(x_hbm.at[src_idx.at[g]], rows[b], sems[b])

    def prefire(kc, base):
        if kc == 0:
            return
        stage_idx(base, 0)
        for b in range(NBUF):
            fire(b, b)

    @pl.when(c == 0)
    def _():
        prefire(k0, s * k0)
    @pl.when(c == 1)
    def _():
        prefire(k1, NS * k0 + s * k1)

    with jax.named_scope("agg_zero"):
        def zinit(i, _):
            for j in range(D // LANES):
                zbuf[i, pl.ds(j * LANES, LANES)] = jnp.zeros(
                    (LANES,), jnp.float32)
            return 0
        lax.fori_loop(0, ZROWS, zinit, 0)
        row0 = s * ROWS_PER_TILE
        for j in range(ROWS_PER_TILE // ZROWS):
            pltpu.sync_copy(zbuf, acc.at[pl.ds(row0 + j * ZROWS, ZROWS)])
        plsc.subcore_barrier()

    def edge_phase(kc, base):
        if kc == 0:
            return
        for h in range(kc // STAGE_CH):
            if h > 0:
                stage_idx(base, h)
                for b in range(NBUF):
                    fire(b, b)
            def ring(go, _):
                for b in range(NBUF):
                    g = go * NBUF + b
                    pltpu.make_async_copy(
                        x_hbm.at[src_idx.at[g]], rows[b], sems[b]).wait()
                    pltpu.sync_copy(rows[b], acc.at[dst_idx.at[g]], add=True)
                    @pl.when(go < STAGE_CH // NBUF - 1)
                    def _():
                        fire(b, g + NBUF)
                return 0
            lax.fori_loop(0, STAGE_CH // NBUF, ring, 0)

    with jax.named_scope("agg_edges"):
        @pl.when(c == 0)
        def _():
            edge_phase(k0, s * k0)
        @pl.when(c == 1)
        def _():
            edge_phase(k1, NS * k0 + s * k1)
        plsc.subcore_barrier()

    with jax.named_scope("agg_out"):
        for j in range(ROWS_PER_TILE // D):
            r = s * ROWS_PER_TILE + j * D
            pltpu.sync_copy(acc.at[pl.ds(r, D)], part_hbm.at[c, pl.ds(r, D)])


def _build_agg(k0, k1):
    scratch = [
        pltpu.VMEM((STAGE_CH, CHUNK), jnp.int32),
        pltpu.VMEM((STAGE_CH, CHUNK), jnp.int32),
        pltpu.VMEM((CHUNK, D), jnp.float32),
        pltpu.VMEM((CHUNK, D), jnp.float32),
        pltpu.VMEM((CHUNK, D), jnp.float32),
        pltpu.VMEM((CHUNK, D), jnp.float32),
        pltpu.VMEM((ZROWS, D), jnp.float32),
        pltpu.VMEM_SHARED((N_PAD, D), jnp.float32),
        pltpu.SemaphoreType.DMA,
        pltpu.SemaphoreType.DMA,
        pltpu.SemaphoreType.DMA,
        pltpu.SemaphoreType.DMA,
    ]
    return pl.kernel(
        functools.partial(_agg_body, k0, k1),
        out_type=(jax.ShapeDtypeStruct((NC, N_PAD, D), jnp.float32),),
        mesh=_mesh(),
        scratch_types=scratch,
        compiler_params=_SC_PARAMS,
        name="sage_agg_sc",
    )


def _cnt_body(dst_hbm, cnt_hbm, dst_idx, cnt_local):
    c = lax.axis_index("c")
    s = lax.axis_index("s")
    k_chunks = dst_hbm.shape[0] // NW
    chunk = dst_hbm.shape[1]
    wid = c * NS + s

    def cinit(i, _):
        for j in range(D // LANES):
            cnt_local[pl.ds(i * D + j * LANES, LANES)] = jnp.zeros(
                (LANES,), jnp.float32)
        return 0
    lax.fori_loop(0, N_PAD // D, cinit, 0)

    pltpu.sync_copy(dst_hbm.at[pl.ds(wid * k_chunks, k_chunks)], dst_idx)
    ones16 = jnp.ones((LANES,), jnp.float32)
    def cbody(k, _):
        for j in range(chunk // LANES):
            idxv = dst_idx[k, pl.ds(j * LANES, LANES)]
            plsc.addupdate_scatter(cnt_local, [idxv], ones16)
        return 0
    lax.fori_loop(0, k_chunks, cbody, 0)
    pltpu.sync_copy(cnt_local, cnt_hbm.at[pl.ds(wid * N_PAD, N_PAD)])


def _build_cnt(k_chunks):
    scratch = [
        pltpu.VMEM((k_chunks, CHUNK), jnp.int32),
        pltpu.VMEM((N_PAD,), jnp.float32),
    ]
    return pl.kernel(
        _cnt_body,
        out_type=(jax.ShapeDtypeStruct((NW * N_PAD,), jnp.float32),),
        mesh=_mesh(),
        scratch_types=scratch,
        compiler_params=_SC_PARAMS,
        name="sage_cnt_sc",
    )


def _tc_body(relu, p_ref, recip_ref, x_ref, wl_ref, wr_ref, b_ref, o_ref):
    mean = (p_ref[0] + p_ref[1]) * recip_ref[...]
    out = (jnp.dot(mean, wl_ref[...], preferred_element_type=jnp.float32)
           + jnp.dot(x_ref[...], wr_ref[...], preferred_element_type=jnp.float32)
           + b_ref[...])
    if relu:
        out = jnp.maximum(out, 0.0)
    o_ref[...] = out


def _tc_layer(part, recip, x, W_l, W_r, b, relu):
    BT = 1024
    return pl.pallas_call(
        functools.partial(_tc_body, relu),
        grid=(N_PAD // BT,),
        in_specs=[
            pl.BlockSpec((NC, BT, D), lambda i: (0, i, 0)),
            pl.BlockSpec((BT, 1), lambda i: (i, 0)),
            pl.BlockSpec((BT, D), lambda i: (i, 0)),
            pl.BlockSpec((D, D), lambda i: (0, 0)),
            pl.BlockSpec((D, D), lambda i: (0, 0)),
            pl.BlockSpec((1, D), lambda i: (0, 0)),
        ],
        out_specs=pl.BlockSpec((BT, D), lambda i: (i, 0)),
        out_shape=jax.ShapeDtypeStruct((N_PAD, D), jnp.float32),
        name=f"sage_dense_tc_{int(relu)}",
    )(part, recip, x, W_l, W_r, b.reshape(1, D))


def kernel(x, edge_index, W1_l, W1_r, b1, W2_l, W2_r, b2):
    n, d = x.shape
    e = edge_index.shape[1]
    src = edge_index[0].astype(jnp.int32)
    dst = edge_index[1].astype(jnp.int32)

    k_min = -(-e // (NS * CHUNK))
    k_min = -(-k_min // STAGE_CH) * STAGE_CH
    k0 = K_SPLIT0
    k1 = max(K_SPLIT1, k_min - k0)
    k_sum = k0 + k1
    k_chunks = k_sum // 2
    e_pad = NS * k_sum * CHUNK
    pad_src = jnp.arange(e_pad - e, dtype=jnp.int32) % jnp.int32(n)
    src_p = jnp.concatenate([src, pad_src])
    dst_p = jnp.concatenate(
        [dst, jnp.full((e_pad - e,), N_PAD - 1, jnp.int32)])
    src2d = src_p.reshape(NS * k_sum, CHUNK)
    dst2d = dst_p.reshape(NS * k_sum, CHUNK)
    x_pad = jnp.pad(x, ((0, N_PAD - n), (0, 0)))

    agg = _build_agg(k0, k1)
    cntk = _build_cnt(k_chunks)

    (cnt_parts,) = cntk(dst2d)
    cnt = cnt_parts.reshape(NW, N_PAD).sum(axis=0)
    recip = (1.0 / jnp.maximum(cnt, 1.0)).reshape(N_PAD, 1)

    (part1,) = agg(x_pad, src2d, dst2d)
    h = _tc_layer(part1, recip, x_pad, W1_l, W1_r, b1, relu=True)
    (part2,) = agg(h, src2d, dst2d)
    out = _tc_layer(part2, recip, h, W2_l, W2_r, b2, relu=False)
    return out[:n, :]


# speedup vs baseline: 10.7237x; 1.1255x over previous
import functools

import jax
import jax.numpy as jnp
from jax import lax
from jax.experimental import pallas as pl
from jax.experimental.pallas import tpu as pltpu
from jax.experimental.pallas import tpu_sc as plsc

NC, NS, LANES = 2, 16, 16
NW = NC * NS
N_PAD = 10240
D = 128
CHUNK = 64
NBUF = 4
STAGE_CH = 16
ROWS_PER_TILE = N_PAD // NS
ZROWS = 64
K_SPLIT0 = 160
K_SPLIT1 = 0

_SC_PARAMS = pltpu.CompilerParams(needs_layout_passes=False)


def _mesh():
    return plsc.VectorSubcoreMesh(core_axis_name="c", subcore_axis_name="s",
                                  num_cores=NC, num_subcores=NS)


def _agg_body(k0, k1, x_hbm, src_hbm, dst_hbm, part_hbm,
              src_idx, dst_idx, r0, r1, r2, r3, zbuf, acc,
              g0, g1, g2, g3):
    c = lax.axis_index("c")
    s = lax.axis_index("s")
    rows = (r0, r1, r2, r3)
    sems = (g0, g1, g2, g3)

    def stage_idx(base, h):
        pltpu.sync_copy(
            src_hbm.at[pl.ds(base + h * STAGE_CH, STAGE_CH)], src_idx)
        pltpu.sync_copy(
            dst_hbm.at[pl.ds(base + h * STAGE_CH, STAGE_CH)], dst_idx)

    def fire(b, g):
        pltpu.async_copy(x_hbm.at[src_idx.at[g]], rows[b], sems[b])

    def prefire(kc, base):
        if kc == 0:
            return
        stage_idx(base, 0)
        for b in range(NBUF):
            fire(b, b)

    @pl.when(c == 0)
    def _():
        prefire(k0, s * k0)
    @pl.when(c == 1)
    def _():
        prefire(k1, NS * k0 + s * k1)

    with jax.named_scope("agg_zero"):
        def zinit(i, _):
            for j in range(D // LANES):
                zbuf[i, pl.ds(j * LANES, LANES)] = jnp.zeros(
                    (LANES,), jnp.float32)
            return 0
        lax.fori_loop(0, ZROWS, zinit, 0)
        row0 = s * ROWS_PER_TILE
        for j in range(ROWS_PER_TILE // ZROWS):
            pltpu.sync_copy(zbuf, acc.at[pl.ds(row0 + j * ZROWS, ZROWS)])
        plsc.subcore_barrier()

    def edge_phase(kc, base):
        if kc == 0:
            return
        for h in range(kc // STAGE_CH):
            if h > 0:
                stage_idx(base, h)
                for b in range(NBUF):
                    fire(b, b)
            def ring(go, _):
                for b in range(NBUF):
                    g = go * NBUF + b
                    pltpu.make_async_copy(
                        x_hbm.at[src_idx.at[g]], rows[b], sems[b]).wait()
                    pltpu.sync_copy(rows[b], acc.at[dst_idx.at[g]], add=True)
                    @pl.when(go < STAGE_CH // NBUF - 1)
                    def _():
                        fire(b, g + NBUF)
                return 0
            lax.fori_loop(0, STAGE_CH // NBUF, ring, 0)

    with jax.named_scope("agg_edges"):
        @pl.when(c == 0)
        def _():
            edge_phase(k0, s * k0)
        @pl.when(c == 1)
        def _():
            edge_phase(k1, NS * k0 + s * k1)
        plsc.subcore_barrier()

    with jax.named_scope("agg_out"):
        for j in range(ROWS_PER_TILE // D):
            r = s * ROWS_PER_TILE + j * D
            pltpu.sync_copy(acc.at[pl.ds(r, D)], part_hbm.at[c, pl.ds(r, D)])


def _build_agg(k0, k1):
    scratch = [
        pltpu.VMEM((STAGE_CH, CHUNK), jnp.int32),
        pltpu.VMEM((STAGE_CH, CHUNK), jnp.int32),
        pltpu.VMEM((CHUNK, D), jnp.float32),
        pltpu.VMEM((CHUNK, D), jnp.float32),
        pltpu.VMEM((CHUNK, D), jnp.float32),
        pltpu.VMEM((CHUNK, D), jnp.float32),
        pltpu.VMEM((ZROWS, D), jnp.float32),
        pltpu.VMEM_SHARED((N_PAD, D), jnp.float32),
        pltpu.SemaphoreType.DMA,
        pltpu.SemaphoreType.DMA,
        pltpu.SemaphoreType.DMA,
        pltpu.SemaphoreType.DMA,
    ]
    return pl.kernel(
        functools.partial(_agg_body, k0, k1),
        out_type=(jax.ShapeDtypeStruct((NC, N_PAD, D), jnp.float32),),
        mesh=_mesh(),
        scratch_types=scratch,
        compiler_params=_SC_PARAMS,
        name="sage_agg_sc",
    )


def _cnt_body(dst_hbm, cnt_hbm, dst_idx, cnt_local):
    c = lax.axis_index("c")
    s = lax.axis_index("s")
    k_chunks = dst_hbm.shape[0] // NW
    chunk = dst_hbm.shape[1]
    wid = c * NS + s

    def cinit(i, _):
        for j in range(D // LANES):
            cnt_local[pl.ds(i * D + j * LANES, LANES)] = jnp.zeros(
                (LANES,), jnp.float32)
        return 0
    lax.fori_loop(0, N_PAD // D, cinit, 0)

    pltpu.sync_copy(dst_hbm.at[pl.ds(wid * k_chunks, k_chunks)], dst_idx)
    ones16 = jnp.ones((LANES,), jnp.float32)
    def cbody(k, _):
        for j in range(chunk // LANES):
            idxv = dst_idx[k, pl.ds(j * LANES, LANES)]
            plsc.addupdate_scatter(cnt_local, [idxv], ones16)
        return 0
    lax.fori_loop(0, k_chunks, cbody, 0)
    pltpu.sync_copy(cnt_local, cnt_hbm.at[pl.ds(wid * N_PAD, N_PAD)])


def _build_cnt(k_chunks):
    scratch = [
        pltpu.VMEM((k_chunks, CHUNK), jnp.int32),
        pltpu.VMEM((N_PAD,), jnp.float32),
    ]
    return pl.kernel(
        _cnt_body,
        out_type=(jax.ShapeDtypeStruct((NW * N_PAD,), jnp.float32),),
        mesh=_mesh(),
        scratch_types=scratch,
        compiler_params=_SC_PARAMS,
        name="sage_cnt_sc",
    )


def _tc_body(relu, p_ref, recip_ref, x_ref, wl_ref, wr_ref, b_ref, o_ref):
    mean = (p_ref[0] + p_ref[1]) * recip_ref[...]
    out = (jnp.dot(mean, wl_ref[...], preferred_element_type=jnp.float32)
           + jnp.dot(x_ref[...], wr_ref[...], preferred_element_type=jnp.float32)
           + b_ref[...])
    if relu:
        out = jnp.maximum(out, 0.0)
    o_ref[...] = out


def _tc_layer(part, recip, x, W_l, W_r, b, relu):
    BT = 1024
    return pl.pallas_call(
        functools.partial(_tc_body, relu),
        grid=(N_PAD // BT,),
        in_specs=[
            pl.BlockSpec((NC, BT, D), lambda i: (0, i, 0)),
            pl.BlockSpec((BT, 1), lambda i: (i, 0)),
            pl.BlockSpec((BT, D), lambda i: (i, 0)),
            pl.BlockSpec((D, D), lambda i: (0, 0)),
            pl.BlockSpec((D, D), lambda i: (0, 0)),
            pl.BlockSpec((1, D), lambda i: (0, 0)),
        ],
        out_specs=pl.BlockSpec((BT, D), lambda i: (i, 0)),
        out_shape=jax.ShapeDtypeStruct((N_PAD, D), jnp.float32),
        name=f"sage_dense_tc_{int(relu)}",
    )(part, recip, x, W_l, W_r, b.reshape(1, D))


def kernel(x, edge_index, W1_l, W1_r, b1, W2_l, W2_r, b2):
    n, d = x.shape
    e = edge_index.shape[1]
    src = edge_index[0].astype(jnp.int32)
    dst = edge_index[1].astype(jnp.int32)

    k_min = -(-e // (NS * CHUNK))
    k_min = -(-k_min // STAGE_CH) * STAGE_CH
    k0 = K_SPLIT0
    k1 = max(K_SPLIT1, k_min - k0)
    k_sum = k0 + k1
    k_chunks = k_sum // 2
    e_pad = NS * k_sum * CHUNK
    pad_src = jnp.arange(e_pad - e, dtype=jnp.int32) % jnp.int32(n)
    src_p = jnp.concatenate([src, pad_src])
    dst_p = jnp.concatenate(
        [dst, jnp.full((e_pad - e,), N_PAD - 1, jnp.int32)])
    src2d = src_p.reshape(NS * k_sum, CHUNK)
    dst2d = dst_p.reshape(NS * k_sum, CHUNK)
    x_pad = jnp.pad(x, ((0, N_PAD - n), (0, 0)))

    agg = _build_agg(k0, k1)
    cntk = _build_cnt(k_chunks)

    (cnt_parts,) = cntk(dst2d)
    cnt = cnt_parts.reshape(NW, N_PAD).sum(axis=0)
    recip = (1.0 / jnp.maximum(cnt, 1.0)).reshape(N_PAD, 1)

    (part1,) = agg(x_pad, src2d, dst2d)
    h = _tc_layer(part1, recip, x_pad, W1_l, W1_r, b1, relu=True)
    (part2,) = agg(h, src2d, dst2d)
    out = _tc_layer(part2, recip, h, W2_l, W2_r, b2, relu=False)
    return out[:n, :]


# speedup vs baseline: 10.7304x; 1.0006x over previous
import functools

import jax
import jax.numpy as jnp
from jax import lax
from jax.experimental import pallas as pl
from jax.experimental.pallas import tpu as pltpu
from jax.experimental.pallas import tpu_sc as plsc

NC, NS, LANES = 2, 16, 16
NW = NC * NS
N_PAD = 10240
D = 128
CHUNK = 64
NBUF = 4
STAGE_CH = 16
ROWS_PER_TILE = N_PAD // NS
ZROWS = 64
K_SPLIT0 = 160
K_SPLIT1 = 0

_SC_PARAMS = pltpu.CompilerParams(needs_layout_passes=False)


def _mesh():
    return plsc.VectorSubcoreMesh(core_axis_name="c", subcore_axis_name="s",
                                  num_cores=NC, num_subcores=NS)


def _agg_body(k0, k1, x_hbm, src_hbm, dst_hbm, part_hbm,
              src_idx, dst_idx, r0, r1, r2, r3, zbuf, acc,
              g0, g1, g2, g3):
    c = lax.axis_index("c")
    s = lax.axis_index("s")
    rows = (r0, r1, r2, r3)
    sems = (g0, g1, g2, g3)

    def stage_idx(base, h):
        pltpu.sync_copy(
            src_hbm.at[pl.ds(base + h * STAGE_CH, STAGE_CH)], src_idx)
        pltpu.sync_copy(
            dst_hbm.at[pl.ds(base + h * STAGE_CH, STAGE_CH)], dst_idx)

    def fire(b, g):
        pltpu.async_copy(x_hbm.at[src_idx.at[g]], rows[b], sems[b])

    def prefire(kc, base):
        if kc == 0:
            return
        stage_idx(base, 0)
        for b in range(NBUF):
            fire(b, b)

    @pl.when(c == 0)
    def _():
        prefire(k0, s * k0)
    @pl.when(c == 1)
    def _():
        prefire(k1, NS * k0 + s * k1)

    with jax.named_scope("agg_zero"):
        def zinit(i, _):
            for j in range(D // LANES):
                zbuf[i, pl.ds(j * LANES, LANES)] = jnp.zeros(
                    (LANES,), jnp.float32)
            return 0
        lax.fori_loop(0, ZROWS, zinit, 0)
        row0 = s * ROWS_PER_TILE
        for j in range(ROWS_PER_TILE // ZROWS):
            pltpu.sync_copy(zbuf, acc.at[pl.ds(row0 + j * ZROWS, ZROWS)])
        plsc.subcore_barrier()

    def edge_phase(kc, base):
        if kc == 0:
            return
        for h in range(kc // STAGE_CH):
            if h > 0:
                stage_idx(base, h)
                for b in range(NBUF):
                    fire(b, b)
            def ring(go, _):
                for b in range(NBUF):
                    g = go * NBUF + b
                    pltpu.make_async_copy(
                        x_hbm.at[src_idx.at[g]], rows[b], sems[b]).wait()
                    pltpu.sync_copy(rows[b], acc.at[dst_idx.at[g]], add=True)
                    @pl.when(go < STAGE_CH // NBUF - 1)
                    def _():
                        fire(b, g + NBUF)
                return 0
            lax.fori_loop(0, STAGE_CH // NBUF, ring, 0)

    with jax.named_scope("agg_edges"):
        @pl.when(c == 0)
        def _():
            edge_phase(k0, s * k0)
        @pl.when(c == 1)
        def _():
            edge_phase(k1, NS * k0 + s * k1)
        plsc.subcore_barrier()

    with jax.named_scope("agg_out"):
        for j in range(ROWS_PER_TILE // D):
            r = s * ROWS_PER_TILE + j * D
            pltpu.sync_copy(acc.at[pl.ds(r, D)], part_hbm.at[c, pl.ds(r, D)])


def _build_agg(k0, k1):
    scratch = [
        pltpu.VMEM((STAGE_CH, CHUNK), jnp.int32),
        pltpu.VMEM((STAGE_CH, CHUNK), jnp.int32),
        pltpu.VMEM((CHUNK, D), jnp.float32),
        pltpu.VMEM((CHUNK, D), jnp.float32),
        pltpu.VMEM((CHUNK, D), jnp.float32),
        pltpu.VMEM((CHUNK, D), jnp.float32),
        pltpu.VMEM((ZROWS, D), jnp.float32),
        pltpu.VMEM_SHARED((N_PAD, D), jnp.float32),
        pltpu.SemaphoreType.DMA,
        pltpu.SemaphoreType.DMA,
        pltpu.SemaphoreType.DMA,
        pltpu.SemaphoreType.DMA,
    ]
    return pl.kernel(
        functools.partial(_agg_body, k0, k1),
        out_type=(jax.ShapeDtypeStruct((NC, N_PAD, D), jnp.float32),),
        mesh=_mesh(),
        scratch_types=scratch,
        compiler_params=_SC_PARAMS,
        name="sage_agg_sc",
    )


def _cnt_body(dst_hbm, cnt_hbm, dst_idx, cnt_local):
    c = lax.axis_index("c")
    s = lax.axis_index("s")
    k_chunks = dst_hbm.shape[0] // NW
    chunk = dst_hbm.shape[1]
    wid = c * NS + s

    def cinit(i, _):
        for j in range(D // LANES):
            cnt_local[pl.ds(i * D + j * LANES, LANES)] = jnp.zeros(
                (LANES,), jnp.float32)
        return 0
    lax.fori_loop(0, N_PAD // D, cinit, 0)

    pltpu.sync_copy(dst_hbm.at[pl.ds(wid * k_chunks, k_chunks)], dst_idx)
    ones16 = jnp.ones((LANES,), jnp.float32)
    def cbody(k, _):
        for j in range(chunk // LANES):
            idxv = dst_idx[k, pl.ds(j * LANES, LANES)]
            plsc.addupdate_scatter(cnt_local, [idxv], ones16)
        return 0
    lax.fori_loop(0, k_chunks, cbody, 0)
    pltpu.sync_copy(cnt_local, cnt_hbm.at[pl.ds(wid * N_PAD, N_PAD)])


def _build_cnt(k_chunks):
    scratch = [
        pltpu.VMEM((k_chunks, CHUNK), jnp.int32),
        pltpu.VMEM((N_PAD,), jnp.float32),
    ]
    return pl.kernel(
        _cnt_body,
        out_type=(jax.ShapeDtypeStruct((NW * N_PAD,), jnp.float32),),
        mesh=_mesh(),
        scratch_types=scratch,
        compiler_params=_SC_PARAMS,
        name="sage_cnt_sc",
    )


def _tc_body(relu, p_ref, recip_ref, x_ref, wl_ref, wr_ref, b_ref, o_ref):
    mean = (p_ref[0] + p_ref[1]) * recip_ref[...]
    out = (jnp.dot(mean, wl_ref[...], preferred_element_type=jnp.float32)
           + jnp.dot(x_ref[...], wr_ref[...], preferred_element_type=jnp.float32)
           + b_ref[...])
    if relu:
        out = jnp.maximum(out, 0.0)
    o_ref[...] = out


def _tc_layer(part, recip, x, W_l, W_r, b, relu):
    BT = 1024
    return pl.pallas_call(
        functools.partial(_tc_body, relu),
        grid=(N_PAD // BT,),
        in_specs=[
            pl.BlockSpec((NC, BT, D), lambda i: (0, i, 0)),
            pl.BlockSpec((BT, 1), lambda i: (i, 0)),
            pl.BlockSpec((BT, D), lambda i: (i, 0)),
            pl.BlockSpec((D, D), lambda i: (0, 0)),
            pl.BlockSpec((D, D), lambda i: (0, 0)),
            pl.BlockSpec((1, D), lambda i: (0, 0)),
        ],
        out_specs=pl.BlockSpec((BT, D), lambda i: (i, 0)),
        out_shape=jax.ShapeDtypeStruct((N_PAD, D), jnp.float32),
        name=f"sage_dense_tc_{int(relu)}",
    )(part, recip, x, W_l, W_r, b.reshape(1, D))


def kernel(x, edge_index, W1_l, W1_r, b1, W2_l, W2_r, b2):
    n, d = x.shape
    e = edge_index.shape[1]
    src = edge_index[0].astype(jnp.int32)
    dst = edge_index[1].astype(jnp.int32)

    k_min = -(-e // (NS * CHUNK))
    k_min = -(-k_min // STAGE_CH) * STAGE_CH
    k0 = K_SPLIT0
    k1 = max(K_SPLIT1, k_min - k0)
    k_sum = k0 + k1
    k_chunks = k_sum // 2
    e_pad = NS * k_sum * CHUNK
    pad_src = jnp.arange(e_pad - e, dtype=jnp.int32) % jnp.int32(n)
    src_p = jnp.concatenate([src, pad_src])
    dst_p = jnp.concatenate(
        [dst, jnp.full((e_pad - e,), N_PAD - 1, jnp.int32)])
    src2d = src_p.reshape(NS * k_sum, CHUNK)
    dst2d = dst_p.reshape(NS * k_sum, CHUNK)
    x_pad = jnp.pad(x, ((0, N_PAD - n), (0, 0)))

    agg = _build_agg(k0, k1)
    cntk = _build_cnt(k_chunks)

    (part1,) = agg(x_pad, src2d, dst2d)
    (cnt_parts,) = cntk(dst2d)
    cnt = cnt_parts.reshape(NW, N_PAD).sum(axis=0)
    recip = (1.0 / jnp.maximum(cnt, 1.0)).reshape(N_PAD, 1)
    h = _tc_layer(part1, recip, x_pad, W1_l, W1_r, b1, relu=True)
    (part2,) = agg(h, src2d, dst2d)
    out = _tc_layer(part2, recip, h, W2_l, W2_r, b2, relu=False)
    return out[:n, :]
